# baseline (speedup 1.0000x reference)
import functools

import jax
import jax.numpy as jnp
from jax import lax
from jax.experimental import pallas as pl
from jax.experimental.pallas import tpu as pltpu
from jax.experimental.pallas import tpu_sc as plsc

MODEL_DIM = 64
SCALE = 8.0

NUM_CORES = 2
NUM_SUBCORES = 16
NUM_WORKERS = NUM_CORES * NUM_SUBCORES

N_ROWS = 4096 * 200
ROWS_PER_WORKER = N_ROWS // NUM_WORKERS
CHUNK = 512
N_CHUNKS = ROWS_PER_WORKER // CHUNK
LANES = 16


@functools.partial(
    pl.kernel,
    out_type=jax.ShapeDtypeStruct((N_ROWS, MODEL_DIM), jnp.float32),
    mesh=plsc.VectorSubcoreMesh(core_axis_name="c", subcore_axis_name="s"),
    compiler_params=pltpu.CompilerParams(use_tc_tiling_on_sc=False),
    scratch_types=[
        pltpu.VMEM((ROWS_PER_WORKER,), jnp.int32),
        pltpu.VMEM((CHUNK, MODEL_DIM), jnp.float32),
        pltpu.VMEM((CHUNK, MODEL_DIM), jnp.float32),
        pltpu.SemaphoreType.DMA,
        pltpu.SemaphoreType.DMA,
    ],
)
def _emb_lookup(table_hbm, idx_hbm, out_hbm, idx_v, buf0, buf1, sem0, sem1):
    wid = lax.axis_index("s") * NUM_CORES + lax.axis_index("c")
    base = wid * ROWS_PER_WORKER
    pltpu.sync_copy(idx_hbm.at[pl.ds(base, ROWS_PER_WORKER)], idx_v)

    def gather(c, buf, sem):
        return pltpu.async_copy(
            table_hbm.at[idx_v.at[pl.ds(c * CHUNK, CHUNK)]], buf, sem)

    def scale(buf):
        def row_body(r, _):
            for c in range(MODEL_DIM // LANES):
                sl = pl.ds(c * LANES, LANES)
                buf[r, sl] = buf[r, sl] * SCALE
            return 0
        lax.fori_loop(0, CHUNK, row_body, 0)

    def writeback(c, buf):
        pltpu.sync_copy(buf, out_hbm.at[pl.ds(base + c * CHUNK, CHUNK)])

    def body(i, _):
        c0 = 2 * i
        c1 = c0 + 1
        h0 = gather(c0, buf0, sem0)
        h1 = gather(c1, buf1, sem1)
        h0.wait()
        scale(buf0)
        writeback(c0, buf0)
        h1.wait()
        scale(buf1)
        writeback(c1, buf1)
        return 0

    lax.fori_loop(0, N_CHUNKS // 2, body, 0)


def kernel(inputs, embeddings):
    idx = inputs.reshape(-1).astype(jnp.int32)
    out = _emb_lookup(embeddings, idx)
    return out.reshape(inputs.shape + (MODEL_DIM,))


# speedup vs baseline: 1.0953x; 1.0953x over previous
import functools

import jax
import jax.numpy as jnp
from jax import lax
from jax.experimental import pallas as pl
from jax.experimental.pallas import tpu as pltpu
from jax.experimental.pallas import tpu_sc as plsc

MODEL_DIM = 64
SCALE = 8.0

NUM_CORES = 2
NUM_SUBCORES = 16
NUM_WORKERS = NUM_CORES * NUM_SUBCORES

N_ROWS = 4096 * 200
ROWS_PER_WORKER = N_ROWS // NUM_WORKERS
CHUNK = 512
N_CHUNKS = ROWS_PER_WORKER // CHUNK
LANES = 16
VECS_PER_ROW = MODEL_DIM // LANES
ROWS_PER_STEP = 4


@functools.partial(
    pl.kernel,
    out_type=jax.ShapeDtypeStruct((N_ROWS, MODEL_DIM), jnp.float32),
    mesh=plsc.VectorSubcoreMesh(core_axis_name="c", subcore_axis_name="s"),
    compiler_params=pltpu.CompilerParams(use_tc_tiling_on_sc=False),
    scratch_types=[
        pltpu.VMEM((ROWS_PER_WORKER,), jnp.int32),
        pltpu.VMEM((CHUNK, MODEL_DIM), jnp.float32),
        pltpu.VMEM((CHUNK, MODEL_DIM), jnp.float32),
        pltpu.SemaphoreType.DMA,
        pltpu.SemaphoreType.DMA,
        pltpu.SemaphoreType.DMA,
        pltpu.SemaphoreType.DMA,
    ],
)
def _emb_lookup(table_hbm, idx_hbm, out_hbm, idx_v, buf0, buf1,
                gsem0, gsem1, wsem0, wsem1):
    wid = lax.axis_index("s") * NUM_CORES + lax.axis_index("c")
    base = wid * ROWS_PER_WORKER
    pltpu.sync_copy(idx_hbm.at[pl.ds(base, ROWS_PER_WORKER)], idx_v)

    def gather(c, buf, sem):
        pltpu.async_copy(table_hbm.at[idx_v.at[pl.ds(c * CHUNK, CHUNK)]],
                         buf, sem)

    def wait_gather(buf, sem):
        pltpu.make_async_copy(table_hbm.at[idx_v.at[pl.ds(0, CHUNK)]],
                              buf, sem).wait()

    def scale(buf):
        def step(s, _):
            r0 = s * ROWS_PER_STEP
            for dr in range(ROWS_PER_STEP):
                for c in range(VECS_PER_ROW):
                    sl = pl.ds(c * LANES, LANES)
                    buf[r0 + dr, sl] = buf[r0 + dr, sl] * SCALE
            return 0
        lax.fori_loop(0, CHUNK // ROWS_PER_STEP, step, 0, unroll=2)

    def writeback(c, buf, sem):
        pltpu.async_copy(buf, out_hbm.at[pl.ds(base + c * CHUNK, CHUNK)], sem)

    def wait_writeback(buf, sem):
        pltpu.make_async_copy(buf, out_hbm.at[pl.ds(0, CHUNK)], sem).wait()

    gather(0, buf0, gsem0)
    gather(1, buf1, gsem1)

    def body(i, _):
        c0 = 2 * i
        c1 = c0 + 1
        wait_gather(buf0, gsem0)
        scale(buf0)
        writeback(c0, buf0, wsem0)
        wait_gather(buf1, gsem1)
        scale(buf1)
        writeback(c1, buf1, wsem1)

        @pl.when(c0 + 2 < N_CHUNKS)
        def _():
            wait_writeback(buf0, wsem0)
            gather(c0 + 2, buf0, gsem0)
            wait_writeback(buf1, wsem1)
            gather(c1 + 2, buf1, gsem1)
        return 0

    lax.fori_loop(0, N_CHUNKS // 2, body, 0)
    wait_writeback(buf0, wsem0)
    wait_writeback(buf1, wsem1)


def kernel(inputs, embeddings):
    idx = inputs.reshape(-1).astype(jnp.int32)
    out = _emb_lookup(embeddings, idx)
    return out.reshape(inputs.shape + (MODEL_DIM,))
